# baseline (speedup 1.0000x reference)
import functools

import numpy as np
import jax
import jax.numpy as jnp
from jax import lax
from jax.experimental import pallas as pl
from jax.experimental.pallas import tpu as pltpu
from jax.experimental.pallas import tpu_sc as plsc

N = 8192
L = 16
V = N // L
H = 4
VH = V // H
R = 64
NW = 32
RPW = R // NW
NBINS = 256
TBL = NBINS * L
HIST = H * TBL

_I32MIN = np.int32(-2147483648)

_GDN = lax.GatherDimensionNumbers(
    offset_dims=(), collapsed_slice_dims=(0,), start_index_map=(0,)
)


def _bcast_last(x):
    idx = jnp.full((L, 1), L - 1, jnp.int32)
    return lax.gather(x, idx, _GDN, (1,),
                      mode=lax.GatherScatterMode.PROMISE_IN_BOUNDS)


def _to_sortable(f):
    u = lax.bitcast_convert_type(f, jnp.int32)
    return u ^ (lax.shift_right_arithmetic(u, 31) | _I32MIN)


def _from_sortable(s):
    return lax.bitcast_convert_type(
        s ^ (lax.shift_right_arithmetic(~s, 31) | _I32MIN), jnp.float32)


def _sort_row(src_f32, ka, kb, hists):
    ones = jnp.ones((L,), jnp.int32)
    lane = lax.iota(jnp.int32, L)
    zeros16 = jnp.zeros((L,), jnp.int32)

    for p in range(4):
        src, dst = (ka, kb) if p % 2 == 0 else (kb, ka)
        shift = 8 * p

        def zero_body(d, _):
            for h in range(H):
                hists[h][pl.ds(d * L, L)] = zeros16
            return 0

        lax.fori_loop(0, NBINS, zero_body, 0, unroll=8)

        if p == 0:
            def count_body(i, _):
                fs = [src_f32[pl.ds(h * (VH * L) + i * L, L)]
                      for h in range(H)]
                ks = [_to_sortable(f) for f in fs]
                for h in range(H):
                    ka[pl.ds(h * (VH * L) + i * L, L)] = ks[h]
                idxs = [(k & 0xFF) * L + lane for k in ks]
                for h in range(H):
                    plsc.addupdate_scatter(hists[h], [idxs[h]], ones)
                return 0
        else:
            def count_body(i, _):
                ks = [src[pl.ds(h * (VH * L) + i * L, L)] for h in range(H)]
                idxs = [(lax.shift_right_logical(k, shift) & 0xFF) * L + lane
                        for k in ks]
                for h in range(H):
                    plsc.addupdate_scatter(hists[h], [idxs[h]], ones)
                return 0

        lax.fori_loop(0, VH, count_body, 0, unroll=4)

        DB = 4

        def scan_body(dd, carry_v):
            base = dd * DB
            rows = [[hists[h][pl.ds((base + q) * L, L)] for h in range(H)]
                    for q in range(DB)]
            ts = [(r[0] + r[1]) + (r[2] + r[3]) for r in rows]
            css = [plsc.cumsum(t) for t in ts]
            bcs = [_bcast_last(cs) for cs in css]
            for q in range(DB):
                start = css[q] - ts[q] + carry_v
                for h in range(H):
                    hists[h][pl.ds((base + q) * L, L)] = start
                    start = start + rows[q][h]
                carry_v = carry_v + bcs[q]
            return carry_v

        lax.fori_loop(0, NBINS // DB, scan_body, zeros16)

        last = p == 3

        def scatter_body(i, _):
            ks = [src[pl.ds(h * (VH * L) + i * L, L)] for h in range(H)]
            idxs = [(lax.shift_right_logical(k, shift) & 0xFF) * L + lane
                    for k in ks]
            rs = [plsc.load_gather(hists[h], [idxs[h]]) for h in range(H)]
            for h in range(H):
                plsc.addupdate_scatter(hists[h], [idxs[h]], ones)
            if last:
                poss = rs
            else:
                poss = [lax.shift_left(r & (V - 1), 4)
                        + lax.shift_right_logical(r, 9) for r in rs]
            for h in range(H):
                plsc.store_scatter(dst, [poss[h]], ks[h])
            return 0

        lax.fori_loop(0, VH, scatter_body, 0, unroll=4)


def _body(input_hbm, target_hbm, out_hbm, stage, stage2, ua, ub, va, vb,
          h0, h1, h2, h3, accv):
    wid = lax.axis_index("s") * 2 + lax.axis_index("c")

    def row_body(rr, accs):
        row = wid * RPW + rr
        pltpu.sync_copy(input_hbm.at[row], stage)
        _sort_row(stage, ua, ub, (h0, h1, h2, h3))
        pltpu.sync_copy(target_hbm.at[row], stage2)
        _sort_row(stage2, va, vb, (h0, h1, h2, h3))

        def diff_body(i, a):
            out = []
            for h in range(H):
                fu = _from_sortable(ua[pl.ds(h * (VH * L) + i * L, L)])
                fv = _from_sortable(va[pl.ds(h * (VH * L) + i * L, L)])
                out.append(a[h] + jnp.abs(fu - fv))
            return tuple(out)

        return lax.fori_loop(0, VH, diff_body, accs, unroll=4)

    z = jnp.zeros((L,), jnp.float32)
    accs = lax.fori_loop(0, RPW, row_body, (z, z, z, z))
    accv[...] = accs[0] + accs[1] + accs[2] + accs[3]
    pltpu.sync_copy(accv, out_hbm.at[wid])


@jax.jit
def kernel(input, target):
    mesh = plsc.VectorSubcoreMesh(
        core_axis_name="c", subcore_axis_name="s", num_cores=2, num_subcores=16
    )
    partials = pl.kernel(
        _body,
        mesh=mesh,
        compiler_params=pltpu.CompilerParams(needs_layout_passes=False),
        out_type=jax.ShapeDtypeStruct((NW, L), jnp.float32),
        scratch_types=[
            pltpu.VMEM((N,), jnp.float32),
            pltpu.VMEM((N,), jnp.float32),
            pltpu.VMEM((N,), jnp.int32),
            pltpu.VMEM((N,), jnp.int32),
            pltpu.VMEM((N,), jnp.int32),
            pltpu.VMEM((N,), jnp.int32),
            pltpu.VMEM((TBL,), jnp.int32),
            pltpu.VMEM((TBL,), jnp.int32),
            pltpu.VMEM((TBL,), jnp.int32),
            pltpu.VMEM((TBL,), jnp.int32),
            pltpu.VMEM((L,), jnp.float32),
        ],
    )(input, target)
    return jnp.sum(partials) * (1.0 / (N * R))


# speedup vs baseline: 1.0290x; 1.0290x over previous
import functools

import numpy as np
import jax
import jax.numpy as jnp
from jax import lax
from jax.experimental import pallas as pl
from jax.experimental.pallas import tpu as pltpu
from jax.experimental.pallas import tpu_sc as plsc

N = 8192
L = 16
V = N // L
H = 4
VH = V // H
R = 64
NW = 32
RPW = R // NW
NBINS = 256
TBL = NBINS * L
HIST = H * TBL

_I32MIN = np.int32(-2147483648)

_GDN = lax.GatherDimensionNumbers(
    offset_dims=(), collapsed_slice_dims=(0,), start_index_map=(0,)
)


def _bcast_last(x):
    idx = jnp.full((L, 1), L - 1, jnp.int32)
    return lax.gather(x, idx, _GDN, (1,),
                      mode=lax.GatherScatterMode.PROMISE_IN_BOUNDS)


def _to_sortable(f):
    u = lax.bitcast_convert_type(f, jnp.int32)
    return u ^ (lax.shift_right_arithmetic(u, 31) | _I32MIN)


def _from_sortable(s):
    return lax.bitcast_convert_type(
        s ^ (lax.shift_right_arithmetic(~s, 31) | _I32MIN), jnp.float32)


def _sort_row(src_f32, ka, kb, hists):
    ones = jnp.ones((L,), jnp.int32)
    lane = lax.iota(jnp.int32, L)
    zeros16 = jnp.zeros((L,), jnp.int32)

    for p in range(4):
        src, dst = (ka, kb) if p % 2 == 0 else (kb, ka)
        shift = 8 * p

        def zero_body(d, _):
            for h in range(H):
                hists[h][pl.ds(d * L, L)] = zeros16
            return 0

        lax.fori_loop(0, NBINS, zero_body, 0, unroll=4)

        if p == 0:
            def count_body(i, _):
                fs = [src_f32[pl.ds(h * (VH * L) + i * L, L)]
                      for h in range(H)]
                ks = [_to_sortable(f) for f in fs]
                for h in range(H):
                    ka[pl.ds(h * (VH * L) + i * L, L)] = ks[h]
                idxs = [(k & 0xFF) * L + lane for k in ks]
                for h in range(H):
                    plsc.addupdate_scatter(hists[h], [idxs[h]], ones)
                return 0
        else:
            def count_body(i, _):
                ks = [src[pl.ds(h * (VH * L) + i * L, L)] for h in range(H)]
                idxs = [(lax.shift_right_logical(k, shift) & 0xFF) * L + lane
                        for k in ks]
                for h in range(H):
                    plsc.addupdate_scatter(hists[h], [idxs[h]], ones)
                return 0

        lax.fori_loop(0, VH, count_body, 0, unroll=4)

        DB = 8

        def scan_body(dd, carry_v):
            base = dd * DB
            rows = [[hists[h][pl.ds((base + q) * L, L)] for h in range(H)]
                    for q in range(DB)]
            ts = [(r[0] + r[1]) + (r[2] + r[3]) for r in rows]
            css = [plsc.cumsum(t) for t in ts]
            bcs = [_bcast_last(cs) for cs in css]
            for q in range(DB):
                start = css[q] - ts[q] + carry_v
                for h in range(H):
                    hists[h][pl.ds((base + q) * L, L)] = start
                    start = start + rows[q][h]
                carry_v = carry_v + bcs[q]
            return carry_v

        lax.fori_loop(0, NBINS // DB, scan_body, zeros16)

        last = p == 3

        def scatter_body(i, _):
            ks = [src[pl.ds(h * (VH * L) + i * L, L)] for h in range(H)]
            idxs = [(lax.shift_right_logical(k, shift) & 0xFF) * L + lane
                    for k in ks]
            rs = [plsc.load_gather(hists[h], [idxs[h]]) for h in range(H)]
            for h in range(H):
                plsc.addupdate_scatter(hists[h], [idxs[h]], ones)
            if last:
                poss = rs
            else:
                poss = [lax.shift_left(r & (V - 1), 4)
                        + lax.shift_right_logical(r, 9) for r in rs]
            for h in range(H):
                plsc.store_scatter(dst, [poss[h]], ks[h])
            return 0

        lax.fori_loop(0, VH, scatter_body, 0, unroll=2)


def _body(input_hbm, target_hbm, out_hbm, stage, ua, ub, va, vb,
          h0, h1, h2, h3, accv):
    wid = lax.axis_index("s") * 2 + lax.axis_index("c")

    def row_body(rr, accs):
        row = wid * RPW + rr
        pltpu.sync_copy(input_hbm.at[row], stage)
        _sort_row(stage, ua, ub, (h0, h1, h2, h3))
        pltpu.sync_copy(target_hbm.at[row], stage)
        _sort_row(stage, va, vb, (h0, h1, h2, h3))

        def diff_body(i, a):
            out = []
            for h in range(H):
                fu = _from_sortable(ua[pl.ds(h * (VH * L) + i * L, L)])
                fv = _from_sortable(va[pl.ds(h * (VH * L) + i * L, L)])
                out.append(a[h] + jnp.abs(fu - fv))
            return tuple(out)

        return lax.fori_loop(0, VH, diff_body, accs, unroll=2)

    z = jnp.zeros((L,), jnp.float32)
    accs = lax.fori_loop(0, RPW, row_body, (z, z, z, z))
    accv[...] = accs[0] + accs[1] + accs[2] + accs[3]
    pltpu.sync_copy(accv, out_hbm.at[wid])


@jax.jit
def kernel(input, target):
    mesh = plsc.VectorSubcoreMesh(
        core_axis_name="c", subcore_axis_name="s", num_cores=2, num_subcores=16
    )
    partials = pl.kernel(
        _body,
        mesh=mesh,
        compiler_params=pltpu.CompilerParams(needs_layout_passes=False),
        out_type=jax.ShapeDtypeStruct((NW, L), jnp.float32),
        scratch_types=[
            pltpu.VMEM((N,), jnp.float32),
            pltpu.VMEM((N,), jnp.int32),
            pltpu.VMEM((N,), jnp.int32),
            pltpu.VMEM((N,), jnp.int32),
            pltpu.VMEM((N,), jnp.int32),
            pltpu.VMEM((TBL,), jnp.int32),
            pltpu.VMEM((TBL,), jnp.int32),
            pltpu.VMEM((TBL,), jnp.int32),
            pltpu.VMEM((TBL,), jnp.int32),
            pltpu.VMEM((L,), jnp.float32),
        ],
    )(input, target)
    return jnp.sum(partials) * (1.0 / (N * R))
